# Initial kernel scaffold; baseline (speedup 1.0000x reference)
import jax
import jax.numpy as jnp
from jax.experimental import pallas as pl


def kernel(x, edge_index, lin1_W, lin1_b, conv_lW, conv_lb, conv_rW, conv_rb, bn_gamma, bn_beta, lin2_W, lin2_b):
    raise NotImplementedError("write your pallas kernel here")


# speedup vs baseline: 6.5536x; 6.5536x over previous
import functools

import jax
import jax.numpy as jnp
from jax import lax
from jax.experimental import pallas as pl
from jax.experimental.pallas import tpu as pltpu
from jax.experimental.pallas import tpu_sc as plsc

BN_EPS = 1e-5

_NC = 2
_NS = 16
_LN = 16
_NW = _NC * _NS

_CW = 128


def _ceil_div(a, b):
    return (a + b - 1) // b


@functools.lru_cache(maxsize=None)
def _make_seg_sum(n, e, h):
    tot = e // _CW
    assert e % _CW == 0
    chmax = _ceil_div(_ceil_div(tot, _NW), 8) * 8
    rpt = _ceil_div(_ceil_div(n, _NS), 8) * 8
    npad = rpt * _NS
    sizes = []
    left = rpt
    while left > 0:
        sizes.append(min(128, left))
        left -= sizes[-1]

    mesh = plsc.VectorSubcoreMesh(
        core_axis_name="c", subcore_axis_name="s",
        num_cores=_NC, num_subcores=_NS)

    grp = 8
    assert chmax % grp == 0
    scratch = [
        pltpu.VMEM((grp, _CW), jnp.int32),
        pltpu.VMEM((grp, _CW), jnp.int32),
        pltpu.VMEM((_CW, h), jnp.float32),
        pltpu.VMEM_SHARED((npad, h), jnp.float32),
        pltpu.SemaphoreType.DMA,
    ]

    def body(h_hbm, src_hbm, dst_hbm, out_hbm, idx_s, idx_d, rows, agg_sh,
             gsem):
        cid = lax.axis_index("c")
        sid = lax.axis_index("s")
        wid = cid * _NS + sid

        zvec = jnp.zeros((_LN,), jnp.float32)

        def zero_row(r, carry):
            for c in range(h // _LN):
                rows[r, pl.ds(c * _LN, _LN)] = zvec
            return carry
        lax.fori_loop(0, 128, zero_row, 0)

        off = 0
        for sz in sizes:
            pltpu.sync_copy(rows.at[pl.ds(0, sz)],
                            agg_sh.at[pl.ds(sid * rpt + off, sz)])
            off += sz

        plsc.subcore_barrier()

        c0 = wid * chmax
        nmine = jnp.clip(tot - wid * chmax, 0, chmax)

        def group(g, carry):
            @pl.when(g * grp < nmine)
            def _():
                pltpu.sync_copy(src_hbm.at[pl.ds(c0 + g * grp, grp)], idx_s)
                pltpu.sync_copy(dst_hbm.at[pl.ds(c0 + g * grp, grp)], idx_d)
                for j in range(grp):
                    @pl.when(g * grp + j < nmine)
                    def _():
                        pltpu.async_copy(h_hbm.at[idx_s.at[j]], rows,
                                         gsem).wait()
                        pltpu.sync_copy(rows, agg_sh.at[idx_d.at[j]],
                                        add=True)
            return carry
        lax.fori_loop(0, chmax // grp, group, 0)

        plsc.subcore_barrier()

        off = 0
        for sz in sizes:
            rbase = sid * rpt + off
            pltpu.sync_copy(agg_sh.at[pl.ds(rbase, sz)], rows.at[pl.ds(0, sz)])
            pltpu.sync_copy(rows.at[pl.ds(0, sz)],
                            out_hbm.at[cid, pl.ds(rbase, sz)])
            off += sz

    return pl.kernel(
        body,
        out_type=jax.ShapeDtypeStruct((_NC, npad, h), jnp.float32),
        mesh=mesh,
        scratch_types=scratch,
    )


def _lin1_body(x_ref, w_ref, b_ref, o_ref):
    acc = jnp.dot(x_ref[...], w_ref[...], preferred_element_type=jnp.float32)
    o_ref[...] = jnp.maximum(acc + b_ref[0:1, :], 0.0)


def _comb_body(sp_ref, cp_ref, r_ref, lw_ref, rw_ref, cb_ref, g_ref, bt_ref,
               o_ref):
    s = sp_ref[0] + sp_ref[1]
    cnt = cp_ref[0, :, 0:1] + cp_ref[1, :, 0:1]
    agg = s / jnp.maximum(cnt, 1.0)
    hh = (jnp.dot(agg, lw_ref[...], preferred_element_type=jnp.float32)
          + jnp.dot(r_ref[...], rw_ref[...], preferred_element_type=jnp.float32)
          + cb_ref[0:1, :])
    o_ref[...] = jnp.maximum(hh * g_ref[0:1, :] + bt_ref[0:1, :], 0.0)


def _comb_last_body(sp_ref, cp_ref, r_ref, lw_ref, rw_ref, cb_ref, g_ref,
                    bt_ref, w2_ref, b2_ref, o_ref):
    s = sp_ref[0] + sp_ref[1]
    cnt = cp_ref[0, :, 0:1] + cp_ref[1, :, 0:1]
    agg = s / jnp.maximum(cnt, 1.0)
    hh = (jnp.dot(agg, lw_ref[...], preferred_element_type=jnp.float32)
          + jnp.dot(r_ref[...], rw_ref[...], preferred_element_type=jnp.float32)
          + cb_ref[0:1, :])
    hbn = hh * g_ref[0:1, :] + bt_ref[0:1, :]
    o_ref[...] = (jnp.dot(hbn, w2_ref[...], preferred_element_type=jnp.float32)
                  + b2_ref[0:1, :])


def _row8(v):
    return jnp.broadcast_to(v[None, :], (8, v.shape[0]))


def kernel(x, edge_index, lin1_W, lin1_b, conv_lW, conv_lb, conv_rW, conv_rb,
           bn_gamma, bn_beta, lin2_W, lin2_b):
    n, d = x.shape
    h = lin1_W.shape[1]
    nlayers = conv_lW.shape[0]
    e = edge_index.shape[1]
    nout = lin2_W.shape[1]

    tot = e // _CW
    chmax = _ceil_div(_ceil_div(tot, _NW), 8) * 8
    padrows = _NW * chmax
    src2d = jnp.zeros((padrows, _CW), jnp.int32).at[:tot].set(
        edge_index[0].reshape(tot, _CW))
    dst2d = jnp.zeros((padrows, _CW), jnp.int32).at[:tot].set(
        edge_index[1].reshape(tot, _CW))

    nb = 1000
    assert n % nb == 0
    grid = (n // nb,)
    full = lambda shape: pl.BlockSpec(shape, lambda i: tuple(0 for _ in shape))
    rowblk = pl.BlockSpec((nb, h), lambda i: (i, 0))

    scale = 1.0 / jnp.sqrt(jnp.float32(1.0) + BN_EPS)

    seg_sum = _make_seg_sum(n, e, h)

    r = pl.pallas_call(
        _lin1_body,
        grid=grid,
        in_specs=[pl.BlockSpec((nb, d), lambda i: (i, 0)), full((d, h)),
                  full((8, h))],
        out_specs=rowblk,
        out_shape=jax.ShapeDtypeStruct((n, h), jnp.float32),
    )(x, lin1_W, _row8(lin1_b))

    cnt_pair = seg_sum(jnp.ones((n, h), jnp.float32), src2d, dst2d)

    for i in range(nlayers):
        s_pair = seg_sum(r, src2d, dst2d)
        cb = _row8(conv_lb[i] + conv_rb[i])
        g = _row8(bn_gamma[i] * scale)
        bt = _row8(bn_beta[i])
        last = i == nlayers - 1
        in_specs = [
            pl.BlockSpec((_NC, nb, h), lambda i_: (0, i_, 0)),
            pl.BlockSpec((_NC, nb, h), lambda i_: (0, i_, 0)),
            rowblk, full((h, h)), full((h, h)),
            full((8, h)), full((8, h)), full((8, h)),
        ]
        args = [s_pair, cnt_pair, r, conv_lW[i], conv_rW[i], cb, g, bt]
        if last:
            in_specs += [full((h, nout)), full((8, nout))]
            args += [lin2_W, _row8(lin2_b)]
        r = pl.pallas_call(
            _comb_last_body if last else _comb_body,
            grid=grid,
            in_specs=in_specs,
            out_specs=pl.BlockSpec((nb, nout if last else h),
                                   lambda i_: (i_, 0)),
            out_shape=jax.ShapeDtypeStruct((n, nout if last else h),
                                           jnp.float32),
        )(*args)
    return r


# speedup vs baseline: 8.9363x; 1.3636x over previous
import functools

import jax
import jax.numpy as jnp
from jax import lax
from jax.experimental import pallas as pl
from jax.experimental.pallas import tpu as pltpu
from jax.experimental.pallas import tpu_sc as plsc

BN_EPS = 1e-5

_NC = 2
_NS = 16
_LN = 16
_NW = _NC * _NS

_CW = 128


def _ceil_div(a, b):
    return (a + b - 1) // b


@functools.lru_cache(maxsize=None)
def _make_seg_sum(n, e, h):
    tot = e // _CW
    assert e % _CW == 0
    chmax = _ceil_div(_ceil_div(tot, _NW), 8) * 8
    rpt = _ceil_div(_ceil_div(n, _NS), 8) * 8
    npad = rpt * _NS
    sizes = []
    left = rpt
    while left > 0:
        sizes.append(min(128, left))
        left -= sizes[-1]

    mesh = plsc.VectorSubcoreMesh(
        core_axis_name="c", subcore_axis_name="s",
        num_cores=_NC, num_subcores=_NS)

    grp = 40
    assert chmax % grp == 0
    ngroups = chmax // grp
    scratch = [
        pltpu.VMEM((grp, _CW), jnp.int32),
        pltpu.VMEM((grp, _CW), jnp.int32),
        pltpu.VMEM((2, _CW, h), jnp.float32),
        pltpu.VMEM_SHARED((npad, h), jnp.float32),
        pltpu.SemaphoreType.DMA,
        pltpu.SemaphoreType.DMA,
    ]

    def body(h_hbm, src_hbm, dst_hbm, out_hbm, idx_s, idx_d, rows, agg_sh,
             gsem0, gsem1):
        cid = lax.axis_index("c")
        sid = lax.axis_index("s")
        wid = cid * _NS + sid
        sems = (gsem0, gsem1)

        zvec = jnp.zeros((_LN,), jnp.float32)

        def zero_row(r, carry):
            for c in range(h // _LN):
                rows[0, r, pl.ds(c * _LN, _LN)] = zvec
            return carry
        lax.fori_loop(0, 128, zero_row, 0)

        off = 0
        for sz in sizes:
            pltpu.sync_copy(rows.at[0, pl.ds(0, sz)],
                            agg_sh.at[pl.ds(sid * rpt + off, sz)])
            off += sz

        plsc.subcore_barrier()

        c0 = wid * chmax
        nmine = jnp.clip(tot - wid * chmax, 0, chmax)

        for g in range(ngroups):
            @pl.when(g * grp < nmine)
            def _():
                pltpu.sync_copy(src_hbm.at[pl.ds(c0 + g * grp, grp)], idx_s)
                pltpu.sync_copy(dst_hbm.at[pl.ds(c0 + g * grp, grp)], idx_d)

                @pl.when(g * grp < nmine)
                def _():
                    pltpu.async_copy(h_hbm.at[idx_s.at[0]], rows.at[0],
                                     sems[0])

                def chunk2(j2, carry):
                    for b in range(2):
                        j = j2 * 2 + b
                        k = g * grp + j

                        @pl.when(k < nmine)
                        def _(j=j, k=k, b=b):
                            pltpu.make_async_copy(
                                h_hbm.at[idx_s.at[j]], rows.at[b],
                                sems[b]).wait()

                            @pl.when(jnp.logical_and(j + 1 < grp,
                                                     k + 1 < nmine))
                            def _():
                                pltpu.async_copy(h_hbm.at[idx_s.at[j + 1]],
                                                 rows.at[1 - b],
                                                 sems[1 - b])
                            pltpu.sync_copy(rows.at[b],
                                            agg_sh.at[idx_d.at[j]],
                                            add=True)
                    return carry
                lax.fori_loop(0, grp // 2, chunk2, 0)

        plsc.subcore_barrier()

        off = 0
        for sz in sizes:
            rbase = sid * rpt + off
            pltpu.sync_copy(agg_sh.at[pl.ds(rbase, sz)],
                            rows.at[0, pl.ds(0, sz)])
            pltpu.sync_copy(rows.at[0, pl.ds(0, sz)],
                            out_hbm.at[cid, pl.ds(rbase, sz)])
            off += sz

    return pl.kernel(
        body,
        out_type=jax.ShapeDtypeStruct((_NC, npad, h), jnp.float32),
        mesh=mesh,
        scratch_types=scratch,
    )


def _lin1_body(x_ref, w_ref, b_ref, o_ref):
    acc = jnp.dot(x_ref[...], w_ref[...], preferred_element_type=jnp.float32)
    o_ref[...] = jnp.maximum(acc + b_ref[0:1, :], 0.0)


def _comb_body(sp_ref, cp_ref, r_ref, lw_ref, rw_ref, cb_ref, g_ref, bt_ref,
               o_ref):
    s = sp_ref[0] + sp_ref[1]
    cnt = cp_ref[0, :, 0:1] + cp_ref[1, :, 0:1]
    agg = s / jnp.maximum(cnt, 1.0)
    hh = (jnp.dot(agg, lw_ref[...], preferred_element_type=jnp.float32)
          + jnp.dot(r_ref[...], rw_ref[...], preferred_element_type=jnp.float32)
          + cb_ref[0:1, :])
    o_ref[...] = jnp.maximum(hh * g_ref[0:1, :] + bt_ref[0:1, :], 0.0)


def _comb_last_body(sp_ref, cp_ref, r_ref, lw_ref, rw_ref, cb_ref, g_ref,
                    bt_ref, w2_ref, b2_ref, o_ref):
    s = sp_ref[0] + sp_ref[1]
    cnt = cp_ref[0, :, 0:1] + cp_ref[1, :, 0:1]
    agg = s / jnp.maximum(cnt, 1.0)
    hh = (jnp.dot(agg, lw_ref[...], preferred_element_type=jnp.float32)
          + jnp.dot(r_ref[...], rw_ref[...], preferred_element_type=jnp.float32)
          + cb_ref[0:1, :])
    hbn = hh * g_ref[0:1, :] + bt_ref[0:1, :]
    o_ref[...] = (jnp.dot(hbn, w2_ref[...], preferred_element_type=jnp.float32)
                  + b2_ref[0:1, :])


def _row8(v):
    return jnp.broadcast_to(v[None, :], (8, v.shape[0]))


def kernel(x, edge_index, lin1_W, lin1_b, conv_lW, conv_lb, conv_rW, conv_rb,
           bn_gamma, bn_beta, lin2_W, lin2_b):
    n, d = x.shape
    h = lin1_W.shape[1]
    nlayers = conv_lW.shape[0]
    e = edge_index.shape[1]
    nout = lin2_W.shape[1]

    tot = e // _CW
    chmax = _ceil_div(_ceil_div(tot, _NW), 8) * 8
    padrows = _NW * chmax
    src2d = jnp.zeros((padrows, _CW), jnp.int32).at[:tot].set(
        edge_index[0].reshape(tot, _CW))
    dst2d = jnp.zeros((padrows, _CW), jnp.int32).at[:tot].set(
        edge_index[1].reshape(tot, _CW))

    nb = 1000
    assert n % nb == 0
    grid = (n // nb,)
    full = lambda shape: pl.BlockSpec(shape, lambda i: tuple(0 for _ in shape))
    rowblk = pl.BlockSpec((nb, h), lambda i: (i, 0))

    scale = 1.0 / jnp.sqrt(jnp.float32(1.0) + BN_EPS)

    seg_sum = _make_seg_sum(n, e, h)

    r = pl.pallas_call(
        _lin1_body,
        grid=grid,
        in_specs=[pl.BlockSpec((nb, d), lambda i: (i, 0)), full((d, h)),
                  full((8, h))],
        out_specs=rowblk,
        out_shape=jax.ShapeDtypeStruct((n, h), jnp.float32),
    )(x, lin1_W, _row8(lin1_b))

    cnt_pair = seg_sum(jnp.ones((n, h), jnp.float32), src2d, dst2d)

    for i in range(nlayers):
        s_pair = seg_sum(r, src2d, dst2d)
        cb = _row8(conv_lb[i] + conv_rb[i])
        g = _row8(bn_gamma[i] * scale)
        bt = _row8(bn_beta[i])
        last = i == nlayers - 1
        in_specs = [
            pl.BlockSpec((_NC, nb, h), lambda i_: (0, i_, 0)),
            pl.BlockSpec((_NC, nb, h), lambda i_: (0, i_, 0)),
            rowblk, full((h, h)), full((h, h)),
            full((8, h)), full((8, h)), full((8, h)),
        ]
        args = [s_pair, cnt_pair, r, conv_lW[i], conv_rW[i], cb, g, bt]
        if last:
            in_specs += [full((h, nout)), full((8, nout))]
            args += [lin2_W, _row8(lin2_b)]
        r = pl.pallas_call(
            _comb_last_body if last else _comb_body,
            grid=grid,
            in_specs=in_specs,
            out_specs=pl.BlockSpec((nb, nout if last else h),
                                   lambda i_: (i_, 0)),
            out_shape=jax.ShapeDtypeStruct((n, nout if last else h),
                                           jnp.float32),
        )(*args)
    return r


# speedup vs baseline: 9.9813x; 1.1169x over previous
import functools

import jax
import jax.numpy as jnp
from jax import lax
from jax.experimental import pallas as pl
from jax.experimental.pallas import tpu as pltpu
from jax.experimental.pallas import tpu_sc as plsc

BN_EPS = 1e-5

_NC = 2
_NS = 16
_LN = 16
_NW = _NC * _NS

_CW = 128


def _ceil_div(a, b):
    return (a + b - 1) // b


@functools.lru_cache(maxsize=None)
def _make_seg_sum(n, e, h, constant_rows=False):
    tot = e // _CW
    assert e % _CW == 0
    chmax = _ceil_div(_ceil_div(tot, _NW), 8) * 8
    rpt = _ceil_div(_ceil_div(n, _NS), 8) * 8
    npad = rpt * _NS
    sizes = []
    left = rpt
    while left > 0:
        sizes.append(min(128, left))
        left -= sizes[-1]

    mesh = plsc.VectorSubcoreMesh(
        core_axis_name="c", subcore_axis_name="s",
        num_cores=_NC, num_subcores=_NS)

    grp = 40
    assert chmax % grp == 0
    ngroups = chmax // grp
    scratch = [
        pltpu.VMEM((grp, _CW), jnp.int32),
        pltpu.VMEM((grp, _CW), jnp.int32),
        pltpu.VMEM((2, _CW, h), jnp.float32),
        pltpu.VMEM_SHARED((npad, h), jnp.float32),
        pltpu.SemaphoreType.DMA,
        pltpu.SemaphoreType.DMA,
    ]

    def body(h_hbm, src_hbm, dst_hbm, out_hbm, idx_s, idx_d, rows, agg_sh,
             gsem0, gsem1):
        cid = lax.axis_index("c")
        sid = lax.axis_index("s")
        wid = cid * _NS + sid
        sems = (gsem0, gsem1)

        zvec = jnp.zeros((_LN,), jnp.float32)

        ovec = jnp.ones((_LN,), jnp.float32)

        def zero_row(r, carry):
            for c in range(h // _LN):
                rows[0, r, pl.ds(c * _LN, _LN)] = zvec
                if constant_rows:
                    rows[1, r, pl.ds(c * _LN, _LN)] = ovec
            return carry
        lax.fori_loop(0, 128, zero_row, 0)

        off = 0
        for sz in sizes:
            pltpu.sync_copy(rows.at[0, pl.ds(0, sz)],
                            agg_sh.at[pl.ds(sid * rpt + off, sz)])
            off += sz

        plsc.subcore_barrier()

        c0 = wid * chmax
        nmine = jnp.clip(tot - wid * chmax, 0, chmax)

        for g in range(ngroups):
            @pl.when(g * grp < nmine)
            def _(g=g):
                if not constant_rows:
                    pltpu.sync_copy(src_hbm.at[pl.ds(c0 + g * grp, grp)],
                                    idx_s)
                pltpu.sync_copy(dst_hbm.at[pl.ds(c0 + g * grp, grp)], idx_d)

                if constant_rows:
                    def chunk1(j, carry):
                        k = g * grp + j

                        @pl.when(k < nmine)
                        def _():
                            pltpu.sync_copy(rows.at[1],
                                            agg_sh.at[idx_d.at[j]],
                                            add=True)
                        return carry
                    lax.fori_loop(0, grp, chunk1, 0)
                    return

                @pl.when(g * grp < nmine)
                def _():
                    pltpu.async_copy(h_hbm.at[idx_s.at[0]], rows.at[0],
                                     sems[0])

                def chunk2(j2, carry):
                    for b in range(2):
                        j = j2 * 2 + b
                        k = g * grp + j

                        @pl.when(k < nmine)
                        def _(j=j, k=k, b=b):
                            pltpu.make_async_copy(
                                h_hbm.at[idx_s.at[j]], rows.at[b],
                                sems[b]).wait()

                            @pl.when(jnp.logical_and(j + 1 < grp,
                                                     k + 1 < nmine))
                            def _():
                                pltpu.async_copy(h_hbm.at[idx_s.at[j + 1]],
                                                 rows.at[1 - b],
                                                 sems[1 - b])
                            pltpu.sync_copy(rows.at[b],
                                            agg_sh.at[idx_d.at[j]],
                                            add=True)
                    return carry
                lax.fori_loop(0, grp // 2, chunk2, 0)

        plsc.subcore_barrier()

        off = 0
        for sz in sizes:
            rbase = sid * rpt + off
            pltpu.sync_copy(agg_sh.at[pl.ds(rbase, sz)],
                            rows.at[0, pl.ds(0, sz)])
            pltpu.sync_copy(rows.at[0, pl.ds(0, sz)],
                            out_hbm.at[cid, pl.ds(rbase, sz)])
            off += sz

    return pl.kernel(
        body,
        out_type=jax.ShapeDtypeStruct((_NC, npad, h), jnp.float32),
        mesh=mesh,
        scratch_types=scratch,
    )


def _lin1_body(x_ref, w_ref, b_ref, o_ref):
    acc = jnp.dot(x_ref[...], w_ref[...], preferred_element_type=jnp.float32)
    o_ref[...] = jnp.maximum(acc + b_ref[0:1, :], 0.0)


def _comb_body(sp_ref, cp_ref, r_ref, lw_ref, rw_ref, cb_ref, g_ref, bt_ref,
               o_ref):
    s = sp_ref[0] + sp_ref[1]
    cnt = cp_ref[0, :, 0:1] + cp_ref[1, :, 0:1]
    agg = s / jnp.maximum(cnt, 1.0)
    hh = (jnp.dot(agg, lw_ref[...], preferred_element_type=jnp.float32)
          + jnp.dot(r_ref[...], rw_ref[...], preferred_element_type=jnp.float32)
          + cb_ref[0:1, :])
    o_ref[...] = jnp.maximum(hh * g_ref[0:1, :] + bt_ref[0:1, :], 0.0)


def _comb_last_body(sp_ref, cp_ref, r_ref, lw_ref, rw_ref, cb_ref, g_ref,
                    bt_ref, w2_ref, b2_ref, o_ref):
    s = sp_ref[0] + sp_ref[1]
    cnt = cp_ref[0, :, 0:1] + cp_ref[1, :, 0:1]
    agg = s / jnp.maximum(cnt, 1.0)
    hh = (jnp.dot(agg, lw_ref[...], preferred_element_type=jnp.float32)
          + jnp.dot(r_ref[...], rw_ref[...], preferred_element_type=jnp.float32)
          + cb_ref[0:1, :])
    hbn = hh * g_ref[0:1, :] + bt_ref[0:1, :]
    o_ref[...] = (jnp.dot(hbn, w2_ref[...], preferred_element_type=jnp.float32)
                  + b2_ref[0:1, :])


def _row8(v):
    return jnp.broadcast_to(v[None, :], (8, v.shape[0]))


def kernel(x, edge_index, lin1_W, lin1_b, conv_lW, conv_lb, conv_rW, conv_rb,
           bn_gamma, bn_beta, lin2_W, lin2_b):
    n, d = x.shape
    h = lin1_W.shape[1]
    nlayers = conv_lW.shape[0]
    e = edge_index.shape[1]
    nout = lin2_W.shape[1]

    tot = e // _CW
    chmax = _ceil_div(_ceil_div(tot, _NW), 8) * 8
    padrows = _NW * chmax
    src2d = jnp.zeros((padrows, _CW), jnp.int32).at[:tot].set(
        edge_index[0].reshape(tot, _CW))
    dst2d = jnp.zeros((padrows, _CW), jnp.int32).at[:tot].set(
        edge_index[1].reshape(tot, _CW))

    nb = 1000
    assert n % nb == 0
    grid = (n // nb,)
    full = lambda shape: pl.BlockSpec(shape, lambda i: tuple(0 for _ in shape))
    rowblk = pl.BlockSpec((nb, h), lambda i: (i, 0))

    scale = 1.0 / jnp.sqrt(jnp.float32(1.0) + BN_EPS)

    seg_sum = _make_seg_sum(n, e, h)

    r = pl.pallas_call(
        _lin1_body,
        grid=grid,
        in_specs=[pl.BlockSpec((nb, d), lambda i: (i, 0)), full((d, h)),
                  full((8, h))],
        out_specs=rowblk,
        out_shape=jax.ShapeDtypeStruct((n, h), jnp.float32),
    )(x, lin1_W, _row8(lin1_b))

    cnt_pair = _make_seg_sum(n, e, h, True)(x, src2d, dst2d)

    for i in range(nlayers):
        s_pair = seg_sum(r, src2d, dst2d)
        cb = _row8(conv_lb[i] + conv_rb[i])
        g = _row8(bn_gamma[i] * scale)
        bt = _row8(bn_beta[i])
        last = i == nlayers - 1
        in_specs = [
            pl.BlockSpec((_NC, nb, h), lambda i_: (0, i_, 0)),
            pl.BlockSpec((_NC, nb, h), lambda i_: (0, i_, 0)),
            rowblk, full((h, h)), full((h, h)),
            full((8, h)), full((8, h)), full((8, h)),
        ]
        args = [s_pair, cnt_pair, r, conv_lW[i], conv_rW[i], cb, g, bt]
        if last:
            in_specs += [full((h, nout)), full((8, nout))]
            args += [lin2_W, _row8(lin2_b)]
        r = pl.pallas_call(
            _comb_last_body if last else _comb_body,
            grid=grid,
            in_specs=in_specs,
            out_specs=pl.BlockSpec((nb, nout if last else h),
                                   lambda i_: (i_, 0)),
            out_shape=jax.ShapeDtypeStruct((n, nout if last else h),
                                           jnp.float32),
        )(*args)
    return r


# speedup vs baseline: 10.1233x; 1.0142x over previous
import functools

import jax
import jax.numpy as jnp
from jax import lax
from jax.experimental import pallas as pl
from jax.experimental.pallas import tpu as pltpu
from jax.experimental.pallas import tpu_sc as plsc

BN_EPS = 1e-5

_NC = 2
_NS = 16
_LN = 16
_NW = _NC * _NS

_CW = 128


def _ceil_div(a, b):
    return (a + b - 1) // b


@functools.lru_cache(maxsize=None)
def _make_seg_sum(n, e, h, constant_rows=False):
    tot = e // _CW
    assert e % _CW == 0
    chmax = _ceil_div(_ceil_div(tot, _NW), 8) * 8
    rpt = _ceil_div(_ceil_div(n, _NS), 8) * 8
    npad = rpt * _NS
    sizes = []
    left = rpt
    while left > 0:
        sizes.append(min(128, left))
        left -= sizes[-1]

    mesh = plsc.VectorSubcoreMesh(
        core_axis_name="c", subcore_axis_name="s",
        num_cores=_NC, num_subcores=_NS)

    grp = 40
    assert chmax % grp == 0
    ngroups = chmax // grp
    scratch = [
        pltpu.VMEM((grp, _CW), jnp.int32),
        pltpu.VMEM((grp, _CW), jnp.int32),
        pltpu.VMEM((2, _CW, h), jnp.float32),
        pltpu.VMEM_SHARED((npad, h), jnp.float32),
        pltpu.SemaphoreType.DMA,
        pltpu.SemaphoreType.DMA,
    ]

    def body(h_hbm, src_hbm, dst_hbm, out_hbm, idx_s, idx_d, rows, agg_sh,
             gsem0, gsem1):
        cid = lax.axis_index("c")
        sid = lax.axis_index("s")
        wid = cid * _NS + sid
        sems = (gsem0, gsem1)

        zvec = jnp.zeros((_LN,), jnp.float32)

        ovec = jnp.ones((_LN,), jnp.float32)

        def zero_row(r, carry):
            for c in range(h // _LN):
                rows[0, r, pl.ds(c * _LN, _LN)] = zvec
                if constant_rows:
                    rows[1, r, pl.ds(c * _LN, _LN)] = ovec
            return carry
        lax.fori_loop(0, 128, zero_row, 0)

        off = 0
        for sz in sizes:
            pltpu.sync_copy(rows.at[0, pl.ds(0, sz)],
                            agg_sh.at[pl.ds(sid * rpt + off, sz)])
            off += sz

        plsc.subcore_barrier()

        c0 = wid * chmax
        nmine = jnp.clip(tot - wid * chmax, 0, chmax)

        for g in range(ngroups):
            @pl.when(g * grp < nmine)
            def _(g=g):
                if not constant_rows:
                    pltpu.sync_copy(src_hbm.at[pl.ds(c0 + g * grp, grp)],
                                    idx_s)
                pltpu.sync_copy(dst_hbm.at[pl.ds(c0 + g * grp, grp)], idx_d)

                if constant_rows:
                    def chunk1(j, carry):
                        k = g * grp + j

                        @pl.when(k < nmine)
                        def _():
                            pltpu.sync_copy(rows.at[1],
                                            agg_sh.at[idx_d.at[j]],
                                            add=True)
                        return carry
                    lax.fori_loop(0, grp, chunk1, 0)
                    return

                @pl.when(g * grp < nmine)
                def _():
                    pltpu.async_copy(h_hbm.at[idx_s.at[0]], rows.at[0],
                                     sems[0])

                def chunk2(j2, carry):
                    for b in range(2):
                        j = j2 * 2 + b
                        k = g * grp + j

                        @pl.when(k < nmine)
                        def _(j=j, k=k, b=b):
                            pltpu.make_async_copy(
                                h_hbm.at[idx_s.at[j]], rows.at[b],
                                sems[b]).wait()

                            @pl.when(jnp.logical_and(j + 1 < grp,
                                                     k + 1 < nmine))
                            def _():
                                pltpu.async_copy(h_hbm.at[idx_s.at[j + 1]],
                                                 rows.at[1 - b],
                                                 sems[1 - b])
                            pltpu.sync_copy(rows.at[b],
                                            agg_sh.at[idx_d.at[j]],
                                            add=True)
                    return carry
                lax.fori_loop(0, grp // 2, chunk2, 0)

        plsc.subcore_barrier()

        off = 0
        for sz in sizes:
            rbase = sid * rpt + off
            pltpu.sync_copy(agg_sh.at[pl.ds(rbase, sz)],
                            rows.at[0, pl.ds(0, sz)])
            pltpu.sync_copy(rows.at[0, pl.ds(0, sz)],
                            out_hbm.at[cid, pl.ds(rbase, sz)])
            off += sz

    return pl.kernel(
        body,
        out_type=jax.ShapeDtypeStruct((_NC, npad, h), jnp.float32),
        mesh=mesh,
        scratch_types=scratch,
    )


def _lin1_body(x_ref, w_ref, b_ref, o_ref):
    acc = jnp.dot(x_ref[...], w_ref[...], preferred_element_type=jnp.float32)
    o_ref[...] = jnp.maximum(acc + b_ref[0:1, :], 0.0)


def _comb_body(sp_ref, cp_ref, r_ref, lw_ref, rw_ref, cb_ref, g_ref, bt_ref,
               o_ref):
    s = sp_ref[0] + sp_ref[1]
    cnt = cp_ref[0, :, 0:1] + cp_ref[1, :, 0:1]
    agg = s / jnp.maximum(cnt, 1.0)
    hh = (jnp.dot(agg, lw_ref[...], preferred_element_type=jnp.float32)
          + jnp.dot(r_ref[...], rw_ref[...], preferred_element_type=jnp.float32)
          + cb_ref[0:1, :])
    o_ref[...] = jnp.maximum(hh * g_ref[0:1, :] + bt_ref[0:1, :], 0.0)


def _comb_last_body(sp_ref, cp_ref, r_ref, lw_ref, rw_ref, cb_ref, g_ref,
                    bt_ref, w2_ref, b2_ref, o_ref):
    s = sp_ref[0] + sp_ref[1]
    cnt = cp_ref[0, :, 0:1] + cp_ref[1, :, 0:1]
    agg = s / jnp.maximum(cnt, 1.0)
    hh = (jnp.dot(agg, lw_ref[...], preferred_element_type=jnp.float32)
          + jnp.dot(r_ref[...], rw_ref[...], preferred_element_type=jnp.float32)
          + cb_ref[0:1, :])
    hbn = hh * g_ref[0:1, :] + bt_ref[0:1, :]
    o_ref[...] = (jnp.dot(hbn, w2_ref[...], preferred_element_type=jnp.float32)
                  + b2_ref[0:1, :])


def _row8(v):
    return jnp.broadcast_to(v[None, :], (8, v.shape[0]))


def kernel(x, edge_index, lin1_W, lin1_b, conv_lW, conv_lb, conv_rW, conv_rb,
           bn_gamma, bn_beta, lin2_W, lin2_b):
    n, d = x.shape
    h = lin1_W.shape[1]
    nlayers = conv_lW.shape[0]
    e = edge_index.shape[1]
    nout = lin2_W.shape[1]

    tot = e // _CW
    chmax = _ceil_div(_ceil_div(tot, _NW), 8) * 8
    padrows = _NW * chmax
    src2d = jnp.zeros((padrows, _CW), jnp.int32).at[:tot].set(
        edge_index[0].reshape(tot, _CW))
    dst2d = jnp.zeros((padrows, _CW), jnp.int32).at[:tot].set(
        edge_index[1].reshape(tot, _CW))

    nb = 2000
    assert n % nb == 0
    grid = (n // nb,)
    full = lambda shape: pl.BlockSpec(shape, lambda i: tuple(0 for _ in shape))
    rowblk = pl.BlockSpec((nb, h), lambda i: (i, 0))

    scale = 1.0 / jnp.sqrt(jnp.float32(1.0) + BN_EPS)

    seg_sum = _make_seg_sum(n, e, h)

    r = pl.pallas_call(
        _lin1_body,
        grid=grid,
        in_specs=[pl.BlockSpec((nb, d), lambda i: (i, 0)), full((d, h)),
                  full((8, h))],
        out_specs=rowblk,
        out_shape=jax.ShapeDtypeStruct((n, h), jnp.float32),
    )(x, lin1_W, _row8(lin1_b))

    cnt_pair = _make_seg_sum(n, e, h, True)(x, src2d, dst2d)[:, :, :_LN]

    for i in range(nlayers):
        s_pair = seg_sum(r, src2d, dst2d)
        cb = _row8(conv_lb[i] + conv_rb[i])
        g = _row8(bn_gamma[i] * scale)
        bt = _row8(bn_beta[i])
        last = i == nlayers - 1
        in_specs = [
            pl.BlockSpec((_NC, nb, h), lambda i_: (0, i_, 0)),
            pl.BlockSpec((_NC, nb, _LN), lambda i_: (0, i_, 0)),
            rowblk, full((h, h)), full((h, h)),
            full((8, h)), full((8, h)), full((8, h)),
        ]
        args = [s_pair, cnt_pair, r, conv_lW[i], conv_rW[i], cb, g, bt]
        if last:
            in_specs += [full((h, nout)), full((8, nout))]
            args += [lin2_W, _row8(lin2_b)]
        r = pl.pallas_call(
            _comb_last_body if last else _comb_body,
            grid=grid,
            in_specs=in_specs,
            out_specs=pl.BlockSpec((nb, nout if last else h),
                                   lambda i_: (i_, 0)),
            out_shape=jax.ShapeDtypeStruct((n, nout if last else h),
                                           jnp.float32),
        )(*args)
    return r
